# baseline (speedup 1.0000x reference)
import functools

import jax
import jax.numpy as jnp
from jax import lax
from jax.experimental import pallas as pl
from jax.experimental.pallas import tpu as pltpu
from jax.experimental.pallas import tpu_sc as plsc

N = 100000
D = 128
C = 128
NC, NS = 2, 16
NW = NC * NS
FULL_CHUNKS = N // C
TAIL = N - FULL_CHUNKS * C
STEPS = -(-FULL_CHUNKS // NW)

@functools.cache
def _build():
    mesh = plsc.VectorSubcoreMesh(
        core_axis_name="c", subcore_axis_name="s", num_cores=NC, num_subcores=NS
    )

    @functools.partial(
        pl.kernel,
        out_type=jax.ShapeDtypeStruct((N, D), jnp.float32),
        mesh=mesh,
        scratch_types=[
            pltpu.VMEM((C,), jnp.int32),
            pltpu.VMEM((C, D), jnp.float32),
            pltpu.VMEM((TAIL,), jnp.int32),
            pltpu.VMEM((TAIL, D), jnp.float32),
            pltpu.SemaphoreType.DMA,
        ],
    )
    def _embed_lookup(table_hbm, idx_hbm, out_hbm, idx_v, rows_v, idx_t, rows_t, sem):
        wid = lax.axis_index("s") * NC + lax.axis_index("c")

        def step(i, carry):
            chunk = i * NW + wid

            @pl.when(chunk < FULL_CHUNKS)
            def _():
                base = chunk * C
                pltpu.sync_copy(idx_hbm.at[pl.ds(base, C)], idx_v)
                pltpu.async_copy(table_hbm.at[idx_v], rows_v, sem).wait()
                pltpu.sync_copy(rows_v, out_hbm.at[pl.ds(base, C)])

            return carry

        lax.fori_loop(0, STEPS, step, 0)

        @pl.when(wid == NW - 1)
        def _():
            base = FULL_CHUNKS * C
            pltpu.sync_copy(idx_hbm.at[pl.ds(base, TAIL)], idx_t)
            pltpu.async_copy(table_hbm.at[idx_t], rows_t, sem).wait()
            pltpu.sync_copy(rows_t, out_hbm.at[pl.ds(base, TAIL)])

    return _embed_lookup


def kernel(inputs, embedding):
    idx = inputs.astype(jnp.int32)
    return _build()(embedding, idx)
